# baseline (speedup 1.0000x reference)
import functools

import jax
import jax.numpy as jnp
from jax import lax
from jax.experimental import pallas as pl
from jax.experimental.pallas import tpu as pltpu
from jax.experimental.pallas import tpu_sc as plsc

N_CAT = 26
N_NUM = 13
VOCAB = 100000
EMB = 32
B = 4096
D_EMB = N_CAT * EMB


@functools.cache
def _make_sc_gather(n_rows: int, d: int, dtype=jnp.float32):
    info = plsc.get_sparse_core_info()
    nw = info.num_cores * info.num_subcores
    assert n_rows % (8 * nw) == 0
    r_per_w = n_rows // nw
    mesh = plsc.VectorSubcoreMesh(core_axis_name="c", subcore_axis_name="s")

    @functools.partial(
        pl.kernel,
        mesh=mesh,
        out_type=jax.ShapeDtypeStruct((n_rows, d), dtype),
        scratch_types=[
            pltpu.VMEM((r_per_w,), jnp.int32),
            pltpu.VMEM((r_per_w, d), dtype),
            pltpu.SemaphoreType.DMA,
        ],
        compiler_params=pltpu.CompilerParams(use_tc_tiling_on_sc=False),
    )
    def gather_k(idx_hbm, table_hbm, out_hbm, idx_v, rows_v, sem):
        wid = lax.axis_index("s") * info.num_cores + lax.axis_index("c")
        base = wid * r_per_w
        pltpu.sync_copy(idx_hbm.at[pl.ds(base, r_per_w)], idx_v)
        pltpu.async_copy(table_hbm.at[idx_v], rows_v, sem).wait()
        pltpu.sync_copy(rows_v, out_hbm.at[pl.ds(base, r_per_w)])

    return gather_k


def _mlp_body(emb, xn, w1a, w1b, b1r, w2, b2r, w3, b3r, out):
    h = jnp.dot(emb[...], w1a[...], preferred_element_type=jnp.float32)
    h = h + jnp.dot(xn[...], w1b[...], preferred_element_type=jnp.float32)
    h = jnp.maximum(h + b1r[...], 0.0)
    h = jnp.dot(h, w2[...], preferred_element_type=jnp.float32) + b2r[...]
    h = jnp.maximum(h, 0.0)
    h = jnp.dot(h, w3[...], preferred_element_type=jnp.float32) + b3r[...]
    out[...] = jnp.maximum(h, 0.0)


@functools.cache
def _make_mlp(tb: int):
    grid = (B // tb,)
    return pl.pallas_call(
        _mlp_body,
        grid=grid,
        in_specs=[
            pl.BlockSpec((tb, D_EMB), lambda i: (i, 0)),
            pl.BlockSpec((tb, N_NUM), lambda i: (i, 0)),
            pl.BlockSpec((D_EMB, 512), lambda i: (0, 0)),
            pl.BlockSpec((N_NUM, 512), lambda i: (0, 0)),
            pl.BlockSpec((1, 512), lambda i: (0, 0)),
            pl.BlockSpec((512, 256), lambda i: (0, 0)),
            pl.BlockSpec((1, 256), lambda i: (0, 0)),
            pl.BlockSpec((256, 128), lambda i: (0, 0)),
            pl.BlockSpec((1, 128), lambda i: (0, 0)),
        ],
        out_specs=pl.BlockSpec((tb, 128), lambda i: (i, 0)),
        out_shape=jax.ShapeDtypeStruct((B, 128), jnp.float32),
    )


def kernel(x, tables, W1, b1, W2, b2, W3, b3):
    idx = x[:, :N_CAT].astype(jnp.int32)
    offsets = (jnp.arange(N_CAT, dtype=jnp.int32) * VOCAB)[None, :]
    flat_idx = (idx + offsets).reshape(-1)
    table2d = tables.reshape(N_CAT * VOCAB, EMB).astype(jnp.bfloat16)
    x_num = x[:, N_CAT:]

    emb = _make_sc_gather(B * N_CAT, EMB, jnp.bfloat16)(flat_idx, table2d)
    emb = emb.reshape(B, D_EMB)

    out = _make_mlp(512)(
        emb,
        x_num,
        W1[:D_EMB].astype(jnp.bfloat16),
        W1[D_EMB:],
        b1[None, :],
        W2,
        b2[None, :],
        W3,
        b3[None, :],
    )
    return out


# speedup vs baseline: 1.2174x; 1.2174x over previous
import functools

import jax
import jax.numpy as jnp
from jax import lax
from jax.experimental import pallas as pl
from jax.experimental.pallas import tpu as pltpu
from jax.experimental.pallas import tpu_sc as plsc

N_CAT = 26
N_NUM = 13
VOCAB = 100000
EMB = 32
B = 4096
D_EMB = N_CAT * EMB


@functools.cache
def _make_sc_gather(n_rows: int, d: int, dtype=jnp.float32):
    info = plsc.get_sparse_core_info()
    nw = info.num_cores * info.num_subcores
    assert n_rows % (8 * nw) == 0
    r_per_w = n_rows // nw
    mesh = plsc.VectorSubcoreMesh(core_axis_name="c", subcore_axis_name="s")

    @functools.partial(
        pl.kernel,
        mesh=mesh,
        out_type=jax.ShapeDtypeStruct((n_rows, d), dtype),
        scratch_types=[
            pltpu.VMEM((r_per_w,), jnp.int32),
            pltpu.VMEM((r_per_w, d), dtype),
            pltpu.SemaphoreType.DMA,
        ],
        compiler_params=pltpu.CompilerParams(use_tc_tiling_on_sc=False),
    )
    def gather_k(idx_hbm, table_hbm, out_hbm, idx_v, rows_v, sem):
        wid = lax.axis_index("s") * info.num_cores + lax.axis_index("c")
        base = wid * r_per_w
        pltpu.sync_copy(idx_hbm.at[pl.ds(base, r_per_w)], idx_v)
        pltpu.async_copy(table_hbm.at[idx_v], rows_v, sem).wait()
        pltpu.sync_copy(rows_v, out_hbm.at[pl.ds(base, r_per_w)])

    return gather_k


def _mlp_body(emb, xn, w1a, w1b, b1r, w2, b2r, w3, b3r, out):
    h = jnp.dot(emb[...], w1a[...], preferred_element_type=jnp.float32)
    h = h + jnp.dot(xn[...], w1b[...], preferred_element_type=jnp.float32)
    h = jnp.maximum(h + b1r[...], 0.0)
    h = jnp.dot(h, w2[...], preferred_element_type=jnp.float32) + b2r[...]
    h = jnp.maximum(h, 0.0)
    h = jnp.dot(h, w3[...], preferred_element_type=jnp.float32) + b3r[...]
    out[...] = jnp.maximum(h, 0.0)


@functools.cache
def _make_mlp(tb: int):
    grid = (B // tb,)
    return pl.pallas_call(
        _mlp_body,
        grid=grid,
        in_specs=[
            pl.BlockSpec((tb, D_EMB), lambda i: (i, 0)),
            pl.BlockSpec((tb, N_NUM), lambda i: (i, 0)),
            pl.BlockSpec((D_EMB, 512), lambda i: (0, 0)),
            pl.BlockSpec((N_NUM, 512), lambda i: (0, 0)),
            pl.BlockSpec((1, 512), lambda i: (0, 0)),
            pl.BlockSpec((512, 256), lambda i: (0, 0)),
            pl.BlockSpec((1, 256), lambda i: (0, 0)),
            pl.BlockSpec((256, 128), lambda i: (0, 0)),
            pl.BlockSpec((1, 128), lambda i: (0, 0)),
        ],
        out_specs=pl.BlockSpec((tb, 128), lambda i: (i, 0)),
        out_shape=jax.ShapeDtypeStruct((B, 128), jnp.float32),
    )


def kernel(x, tables, W1, b1, W2, b2, W3, b3):
    idx = x[:, :N_CAT].astype(jnp.int32)
    offsets = (jnp.arange(N_CAT, dtype=jnp.int32) * VOCAB)[None, :]
    flat_idx = (idx + offsets).reshape(-1)
    table2d = tables.reshape(N_CAT * VOCAB, EMB)
    x_num = x[:, N_CAT:]

    emb = _make_sc_gather(B * N_CAT, EMB)(flat_idx, table2d)
    emb = emb.reshape(B, D_EMB)

    out = _make_mlp(512)(
        emb,
        x_num,
        W1[:D_EMB],
        W1[D_EMB:],
        b1[None, :],
        W2,
        b2[None, :],
        W3,
        b3[None, :],
    )
    return out
